# Initial kernel scaffold; baseline (speedup 1.0000x reference)
import jax
import jax.numpy as jnp
from jax.experimental import pallas as pl


def kernel(unary, binary, index1, index2):
    raise NotImplementedError("write your pallas kernel here")


# speedup vs baseline: 2.4248x; 2.4248x over previous
import functools

import jax
import jax.numpy as jnp
from jax import lax
from jax.experimental import pallas as pl
from jax.experimental.pallas import tpu as pltpu
from jax.experimental.pallas import tpu_sc as plsc

NC = 2
NS = 16
NW = NC * NS
G = 128


def _sc_join(unary, binary, idx1g, idx2g):
    V, D = unary.shape
    B, E = binary.shape
    NG = idx1g.shape[0]
    W = 2 * D + E
    base_pw = NG // NW
    rem = NG - base_pw * NW

    mesh = plsc.VectorSubcoreMesh(core_axis_name="c", subcore_axis_name="s")

    @functools.partial(
        pl.kernel,
        out_type=jax.ShapeDtypeStruct((B, W), jnp.float32),
        mesh=mesh,
        scratch_types=[
            pltpu.VMEM((G,), jnp.int32),
            pltpu.VMEM((G,), jnp.int32),
            pltpu.VMEM((G, D), jnp.float32),
            pltpu.VMEM((G, D), jnp.float32),
            pltpu.VMEM((G, E), jnp.float32),
            pltpu.SemaphoreType.DMA,
        ],
    )
    def join_kernel(unary_h, binary_h, idx1_h, idx2_h, out_h,
                    i1_v, i2_v, r1_v, r2_v, b_v, sem):
        cid = lax.axis_index("c")
        sid = lax.axis_index("s")
        wid = sid * NC + cid

        def do_group(g):
            row = g * G
            pltpu.sync_copy(idx1_h.at[g], i1_v)
            pltpu.sync_copy(idx2_h.at[g], i2_v)
            c1 = pltpu.async_copy(unary_h.at[i1_v], r1_v, sem)
            c2 = pltpu.async_copy(unary_h.at[i2_v], r2_v, sem)
            pltpu.sync_copy(binary_h.at[pl.ds(row, G)], b_v)
            c1.wait()
            c2.wait()
            pltpu.sync_copy(r1_v, out_h.at[pl.ds(row, G), pl.ds(0, D)])
            pltpu.sync_copy(r2_v, out_h.at[pl.ds(row, G), pl.ds(D, D)])
            pltpu.sync_copy(b_v, out_h.at[pl.ds(row, G), pl.ds(2 * D, E)])

        g0 = wid * base_pw

        @pl.loop(g0, g0 + base_pw)
        def _(g):
            do_group(g)

        @pl.when(wid < rem)
        def _():
            do_group(NW * base_pw + wid)

    return join_kernel(unary, binary, idx1g, idx2g)


def kernel(unary, binary, index1, index2):
    B = index1.shape[0]
    idx1g = index1.reshape(B // G, G)
    idx2g = index2.reshape(B // G, G)
    return _sc_join(unary, binary, idx1g, idx2g)
